# baseline (speedup 1.0000x reference)
import functools

import jax
import jax.numpy as jnp
from jax import lax
from jax.experimental import pallas as pl
from jax.experimental.pallas import tpu as pltpu

_EMBED = 64
_SCALE = 8.0
_V = 23

_BJ = 8
_BL = 2048


def _onehot_body(x_ref, t_ref, o_ref):
    t8 = t_ref[...] * _SCALE
    vals = lax.broadcasted_iota(jnp.int32, (_V, _BL), 0)
    for jj in range(_BJ):
        idx = x_ref[0, jj, :]
        oh = (idx[None, :] == vals).astype(jnp.float32)
        o_ref[jj] = lax.dot_general(
            t8, oh, (((0,), (0,)), ((), ())),
            precision=lax.Precision.HIGHEST,
            preferred_element_type=jnp.float32,
        )


@functools.cache
def _lookup_kernel(nj, ni):
    grid = (nj // _BJ, ni // _BL)
    return pl.pallas_call(
        _onehot_body,
        grid=grid,
        in_specs=[
            pl.BlockSpec((1, _BJ, _BL), lambda j, i: (0, j, i)),
            pl.BlockSpec((_V, _EMBED), lambda j, i: (0, 0)),
        ],
        out_specs=pl.BlockSpec((_BJ, _EMBED, _BL), lambda j, i: (j, 0, i)),
        out_shape=jax.ShapeDtypeStruct((nj, _EMBED, ni), jnp.float32),
    )


def kernel(x, table):
    b, t, _ = x.shape
    xt = jnp.transpose(x, (2, 1, 0))
    outt = _lookup_kernel(t, b)(xt, table)
    return jnp.transpose(outt, (2, 0, 1))


# speedup vs baseline: 2.4445x; 2.4445x over previous
import functools

import jax
import jax.numpy as jnp
from jax import lax
from jax.experimental import pallas as pl
from jax.experimental.pallas import tpu as pltpu

_EMBED = 64
_SCALE = 8.0
_V = 23

_BJ = 8
_BL = 2048


def _onehot_body(x_ref, t_ref, o_ref):
    t8 = t_ref[...] * _SCALE
    vals = lax.broadcasted_iota(jnp.int32, (_V, _BL), 0)
    for jj in range(_BJ):
        idx = x_ref[0, jj, :]
        oh = (idx[None, :] == vals).astype(jnp.float32)
        o_ref[jj] = lax.dot_general(
            t8, oh, (((0,), (0,)), ((), ())),
            preferred_element_type=jnp.float32,
        )


@functools.cache
def _lookup_kernel(nj, ni):
    grid = (nj // _BJ, ni // _BL)
    return pl.pallas_call(
        _onehot_body,
        grid=grid,
        in_specs=[
            pl.BlockSpec((1, _BJ, _BL), lambda j, i: (0, j, i)),
            pl.BlockSpec((_V, _EMBED), lambda j, i: (0, 0)),
        ],
        out_specs=pl.BlockSpec((_BJ, _EMBED, _BL), lambda j, i: (j, 0, i)),
        out_shape=jax.ShapeDtypeStruct((nj, _EMBED, ni), jnp.float32),
    )


def kernel(x, table):
    b, t, _ = x.shape
    xt = jnp.transpose(x, (2, 1, 0))
    outt = _lookup_kernel(t, b)(xt, table)
    return jnp.transpose(outt, (2, 0, 1))
